# baseline (device time: 197614 ns/iter reference)
import jax
import jax.numpy as jnp
from jax import lax
from jax.experimental import pallas as pl
from jax.experimental.pallas import tpu as pltpu

N_DEV = 32
K = 4
S = 4
NSTEP = N_DEV - 1


def kernel(x, w_mat):
    m_total, k_per = x.shape
    _, n_cols = w_mat.shape
    m_per = m_total // N_DEV
    half = n_cols // 2
    seg = half // K

    def body(x_ref, w_ref, out_ref, part_ref,
             sb_r, rb_r, ss_r, rs_r, cr_r,
             sb_l, rb_l, ss_l, rs_l, cr_l):
        my = lax.axis_index("i")
        left = lax.rem(my + N_DEV - 1, N_DEV)
        right = lax.rem(my + 1, N_DEV)

        barrier = pltpu.get_barrier_semaphore()
        for nbr in (left, right):
            pl.semaphore_signal(barrier, inc=1, device_id=(nbr,),
                                device_id_type=pl.DeviceIdType.MESH)
        pl.semaphore_wait(barrier, 2)

        part_ref[...] = jnp.dot(
            x_ref[...].astype(jnp.bfloat16),
            w_ref[...].astype(jnp.bfloat16),
            preferred_element_type=jnp.float32).astype(jnp.bfloat16)

        def partial_seg(c, colslice):
            return part_ref[pl.ds(c * m_per, m_per), colslice]

        dirs = (
            (sb_r, rb_r, ss_r, rs_r, cr_r, right, left, 0),
            (sb_l, rb_l, ss_l, rs_l, cr_l, left, right, half),
        )

        def mk(d, j, s):
            sb, rb, ss, rs, _, dst, _, _ = d
            return pltpu.make_async_remote_copy(
                src_ref=sb.at[j, s % 2],
                dst_ref=rb.at[j, s % S],
                send_sem=ss.at[j, s % 2],
                recv_sem=rs.at[j, s % S],
                device_id=(dst,),
                device_id_type=pl.DeviceIdType.MESH,
            )

        for s in range(NSTEP):
            for di, d in enumerate(dirs):
                sb, rb, ss, rs, cr, dst, ups, lo = d
                c = lax.rem(my - 1 - s + 2 * N_DEV, N_DEV) if di == 0 \
                    else lax.rem(my + 1 + s, N_DEV)
                for j in range(K):
                    cs = slice(lo + j * seg, lo + (j + 1) * seg)
                    if s == 0:
                        msg = partial_seg(c, cs)
                    else:
                        mk(d, j, s - 1).wait_recv()
                        msg = (rb[j, (s - 1) % S].astype(jnp.float32)
                               + partial_seg(c, cs))
                        pl.semaphore_signal(
                            cr.at[j], inc=1, device_id=(ups,),
                            device_id_type=pl.DeviceIdType.MESH)
                    if s >= S:
                        pl.semaphore_wait(cr.at[j], 1)
                    if s >= 2:
                        mk(d, j, s - 2).wait_send()
                    sb[j, s % 2] = msg.astype(jnp.bfloat16)
                    mk(d, j, s).start()

        for d in dirs:
            sb, rb, ss, rs, cr, dst, ups, lo = d
            for j in range(K):
                cs = slice(lo + j * seg, lo + (j + 1) * seg)
                mk(d, j, NSTEP - 1).wait_recv()
                out_ref[:, cs] = (
                    rb[j, (NSTEP - 1) % S].astype(jnp.float32)
                    + partial_seg(my, cs))
                mk(d, j, NSTEP - 2).wait_send()
                mk(d, j, NSTEP - 1).wait_send()
                pl.semaphore_wait(cr.at[j], S - 1)

    return pl.pallas_call(
        body,
        out_shape=jax.ShapeDtypeStruct((m_per, n_cols), jnp.float32),
        in_specs=[pl.BlockSpec(memory_space=pltpu.VMEM)] * 2,
        out_specs=pl.BlockSpec(memory_space=pltpu.VMEM),
        scratch_shapes=[
            pltpu.VMEM((m_total, n_cols), jnp.bfloat16),
            pltpu.VMEM((K, 2, m_per, seg), jnp.bfloat16),
            pltpu.VMEM((K, S, m_per, seg), jnp.bfloat16),
            pltpu.SemaphoreType.DMA((K, 2)),
            pltpu.SemaphoreType.DMA((K, S)),
            pltpu.SemaphoreType.REGULAR((K,)),
            pltpu.VMEM((K, 2, m_per, seg), jnp.bfloat16),
            pltpu.VMEM((K, S, m_per, seg), jnp.bfloat16),
            pltpu.SemaphoreType.DMA((K, 2)),
            pltpu.SemaphoreType.DMA((K, S)),
            pltpu.SemaphoreType.REGULAR((K,)),
        ],
        compiler_params=pltpu.CompilerParams(collective_id=0),
    )(x, w_mat)
